# baseline (speedup 1.0000x reference)
import functools
import math

import jax
import jax.numpy as jnp
from jax import lax
from jax.experimental import pallas as pl
from jax.experimental.pallas import tpu as pltpu
from jax.experimental.pallas import tpu_sc as plsc

EMBED_DIM = 64
NUM_CHARS = 95
SPECTRAL = 8
BATCH = 1024
SEQ = 50
N_IDX = BATCH * SEQ

try:
    _info = plsc.get_sparse_core_info()
    _NC, _NS = _info.num_cores, _info.num_subcores
except Exception:
    _NC, _NS = 2, 16
_NW = _NC * _NS
_PER_W = N_IDX // _NW
_CHUNK = 80
_N_CHUNK = _PER_W // _CHUNK
_WPAD = 128


def _pgather_kernel(w_hbm, idx_hbm, out_hbm, idx_v, rows0, rows1, sem0, sem1):
    wid = lax.axis_index("s") * _NC + lax.axis_index("c")
    base = wid * _PER_W
    pltpu.sync_copy(idx_hbm.at[pl.ds(base, _PER_W)], idx_v)

    rows = (rows0, rows1)
    sems = (sem0, sem1)

    def start(cp, b):
        pltpu.async_copy(
            w_hbm.at[idx_v.at[pl.ds(cp * _CHUNK, _CHUNK)]], rows[b], sems[b])

    start(0, 0)
    start(1, 1)

    def body(g, carry):
        for b in range(2):
            cp = g * 2 + b
            pltpu.make_async_copy(
                w_hbm.at[idx_v.at[pl.ds(cp * _CHUNK, _CHUNK)]], rows[b],
                sems[b]).wait()
            pltpu.sync_copy(rows[b],
                            out_hbm.at[pl.ds(base + cp * _CHUNK, _CHUNK)])

            @pl.when(cp + 2 < _N_CHUNK)
            def _():
                start(cp + 2, b)
        return carry

    lax.fori_loop(0, _N_CHUNK // 2, body, 0)


@functools.cache
def _pgather_call():
    return pl.kernel(
        _pgather_kernel,
        out_type=jax.ShapeDtypeStruct((N_IDX, _WPAD), jnp.float32),
        mesh=plsc.VectorSubcoreMesh(core_axis_name="c", subcore_axis_name="s"),
        scratch_types=[
            pltpu.VMEM((_PER_W,), jnp.int32),
            pltpu.VMEM((_CHUNK, _WPAD), jnp.float32),
            pltpu.VMEM((_CHUNK, _WPAD), jnp.float32),
            pltpu.SemaphoreType.DMA,
            pltpu.SemaphoreType.DMA,
        ],
    )


def _wave_kernel(pos_ref, p_ref, w_ref, r_ref, s_ref, c_ref):
    p = p_ref[0]
    omega = p[0:1, :] * 2.0
    a1 = p[1:2, :]
    a2 = p[2:3, :]
    a3 = p[3:4, :]
    beta = p[4:5, :]
    gamma = 1.0 / (1.0 + jnp.exp(-p[5:6, :]))
    phi = p[6:7, :] * math.pi

    j = lax.broadcasted_iota(jnp.int32, (EMBED_DIM, BATCH), 0).astype(
        jnp.float32)
    theta = omega * j + phi
    s1 = jnp.sin(theta)
    c1 = jnp.cos(theta)
    base = (a1 * s1 + a2 * (2.0 * s1 * c1)
            + a3 * (3.0 - 4.0 * s1 * s1) * s1) * jnp.exp(-gamma * j)
    psl = jnp.sin(pos_ref[pl.program_id(0)].astype(jnp.float32)
                  * (0.1 * math.pi))
    wave = base + (beta * psl) * j

    w_ref[...] = wave[None]
    r_ref[...] = pltpu.roll(wave, 1, axis=0)[None]
    s_ref[...] = jnp.sin(wave)[None]
    c_ref[...] = jnp.cos(wave)[None]


_wave_call = pl.pallas_call(
    _wave_kernel,
    grid=(SEQ,),
    in_specs=[
        pl.BlockSpec(memory_space=pltpu.SMEM),
        pl.BlockSpec((1, SPECTRAL, BATCH), lambda l: (l, 0, 0)),
    ],
    out_specs=[pl.BlockSpec((1, EMBED_DIM, BATCH), lambda l: (l, 0, 0))] * 4,
    out_shape=[jax.ShapeDtypeStruct((SEQ, EMBED_DIM, BATCH), jnp.float32)] * 4,
)


def kernel(indices, positions, W):
    idxflat = indices.T.reshape(N_IDX)
    wpad = jnp.concatenate(
        [W, jnp.zeros((NUM_CHARS, _WPAD - SPECTRAL), W.dtype)], axis=1)
    pg = _pgather_call()(wpad, idxflat)
    pgt = (pg.reshape(SEQ, BATCH, _WPAD)[:, :, :SPECTRAL]
           .transpose(0, 2, 1))
    wt_, rt_, st_, ct_ = _wave_call(positions, pgt)
    psi = [a.transpose(2, 0, 1) for a in (wt_, rt_, st_, ct_)]
    return jnp.stack(psi, axis=-1)
